# Initial kernel scaffold; baseline (speedup 1.0000x reference)
import jax
import jax.numpy as jnp
from jax.experimental import pallas as pl


def kernel(X1, X2):
    raise NotImplementedError("write your pallas kernel here")


# speedup vs baseline: 1.5978x; 1.5978x over previous
import jax
import jax.numpy as jnp
from jax import lax
from jax.experimental import pallas as pl
from jax.experimental.pallas import tpu as pltpu
from jax.experimental.pallas import tpu_sc as plsc

N = 1024
D = 128
M = 100000
K = 32
B = 1024
NB = 98

_NEG = -3e38


def _dist_topk_body(x1_ref, x2_ref, n1_ref, n2_ref, out_ref, val_ref,
                    idx_ref, score_ref):
    j = pl.program_id(0)
    out_ref[...] = jnp.zeros_like(out_ref)

    x1 = x1_ref[...]
    x2 = x2_ref[...]
    g = lax.dot_general(x1, x2, (((1,), (1,)), ((), ())),
                        preferred_element_type=jnp.float32)
    sq = (n1_ref[...] + n2_ref[...]) - 2.0 * g
    dneg = -jnp.sqrt(jnp.maximum(sq, 0.0))
    col = j * B + lax.broadcasted_iota(jnp.int32, (N, B), 1)
    dneg = jnp.where(col < M, dneg, _NEG)
    row = lax.broadcasted_iota(jnp.int32, (N, B), 0)
    fidx = row * M + col

    @pl.when(j == 0)
    def _():
        val_ref[...] = jnp.full((N, K), _NEG, jnp.float32)
        idx_ref[...] = jnp.zeros((N, K), jnp.int32)

    wv = jnp.concatenate([val_ref[...], dneg], axis=1)
    wi = jnp.concatenate([idx_ref[...], fidx], axis=1)
    vs, ids = [], []
    for _ in range(K):
        m = jnp.max(wv, axis=1, keepdims=True)
        eq = wv == m
        ci = jnp.min(jnp.where(eq, wi, jnp.int32(2**31 - 1)), axis=1,
                     keepdims=True)
        vs.append(m)
        ids.append(ci)
        wv = jnp.where(eq & (wi == ci), _NEG, wv)
    val_new = jnp.concatenate(vs, axis=1)
    val_ref[...] = val_new
    idx_ref[...] = jnp.concatenate(ids, axis=1)

    @pl.when(j == NB - 1)
    def _():
        e = jnp.exp(val_new - val_new[:, 0:1])
        score_ref[...] = e / jnp.sum(e, axis=1, keepdims=True)


def _dist_topk(X1, X2, n1, n2T):
    return pl.pallas_call(
        _dist_topk_body,
        grid=(NB,),
        in_specs=[
            pl.BlockSpec((N, D), lambda j: (0, 0)),
            pl.BlockSpec((B, D), lambda j: (j, 0)),
            pl.BlockSpec((N, 1), lambda j: (0, 0)),
            pl.BlockSpec((1, B), lambda j: (0, j)),
        ],
        out_specs=[
            pl.BlockSpec((N, B), lambda j: (0, j)),
            pl.BlockSpec((N, K), lambda j: (0, 0)),
            pl.BlockSpec((N, K), lambda j: (0, 0)),
            pl.BlockSpec((N, K), lambda j: (0, 0)),
        ],
        out_shape=[
            jax.ShapeDtypeStruct((N, M), jnp.float32),
            jax.ShapeDtypeStruct((N, K), jnp.float32),
            jax.ShapeDtypeStruct((N, K), jnp.int32),
            jax.ShapeDtypeStruct((N, K), jnp.float32),
        ],
        compiler_params=pltpu.CompilerParams(
            dimension_semantics=("arbitrary",),
        ),
    )(X1, X2, n1, n2T)


def _sc_scatter(out_flat_ref, scores_flat, fidx_flat):
    info = plsc.get_sparse_core_info()
    nc, ns = info.num_cores, info.num_subcores
    nw = nc * ns
    total = N * K
    ch = total // nw
    assert ch * nw == total
    mesh = plsc.VectorSubcoreMesh(core_axis_name="c", subcore_axis_name="s")

    @pl.kernel(
        out_type=(),
        mesh=mesh,
        scratch_types=[
            pltpu.VMEM((ch,), jnp.float32),
            pltpu.VMEM((ch,), jnp.int32),
        ],
    )
    def scatter_kernel(vals_hbm, idx_hbm, out_hbm, val_v, idx_v):
        wid = lax.axis_index("s") * nc + lax.axis_index("c")
        base = wid * ch
        pltpu.sync_copy(vals_hbm.at[pl.ds(base, ch)], val_v)
        pltpu.sync_copy(idx_hbm.at[pl.ds(base, ch)], idx_v)
        pltpu.sync_copy(val_v, out_hbm.at[idx_v])

    scatter_kernel(scores_flat, fidx_flat, out_flat_ref)


def kernel(X1, X2):
    n1 = jnp.sum(X1 * X1, axis=1, keepdims=True)
    n2T = jnp.sum(X2 * X2, axis=1, keepdims=True).T
    out, _val, idx, score = _dist_topk(X1, X2, n1, n2T)
    out_ref = jax.new_ref(out.reshape(N * M))
    _sc_scatter(out_ref, score.reshape(N * K), idx.reshape(N * K))
    out_final = jax.freeze(out_ref).reshape(N, M)
    return (out_final, score)
